# baseline (speedup 1.0000x reference)
import functools

import jax
import jax.numpy as jnp
from jax.experimental import pallas as pl
from jax.experimental.pallas import tpu as pltpu

C = 200
DIM = 2048
K = 32
MS = 14
P = MS * MS
B = 64


def _masks_kernel(scores_ref, conf_ref, labels_ref, route_ref, fm_ref):
    sc = scores_ref[...]
    mx = jnp.max(sc, axis=1, keepdims=True)
    e = jnp.exp(sc - mx)
    sm = e / jnp.sum(e, axis=1, keepdims=True)
    pred_val = jnp.max(sm, axis=1, keepdims=True)
    io = jax.lax.broadcasted_iota(jnp.int32, (B, C), 1)
    pred_pos = jnp.min(jnp.where(sm == pred_val, io, C), axis=1, keepdims=True)
    lab = labels_ref[...]
    correct = pred_pos == lab
    conf_row = conf_ref[...]
    onehot = (io == lab).astype(jnp.float32)
    conf_l = jnp.sum(onehot * conf_row, axis=1, keepdims=True)
    update_j = correct & (pred_val - conf_l > 0.1)
    forward_j = (correct & (conf_l - pred_val > 0.1)) | (~correct)
    bank_j = conf_l != 0.0
    fmask = forward_j & bank_j & (~update_j)
    bidx = jax.lax.broadcasted_iota(jnp.int32, (B, 1), 0)
    route_ref[...] = jnp.where(update_j, lab, C + bidx).astype(jnp.int32)
    fm_ref[...] = fmask.astype(jnp.int32)


def _match_kernel(labels_ref, feat_ref, bank_ref, pp_ref):
    fv = feat_ref[0]
    bank = bank_ref[0]
    cn = jnp.sqrt(jnp.sum(fv * fv, axis=0, keepdims=True))
    fn = fv / jnp.maximum(cn, 1e-8)
    bn_n = jnp.sqrt(jnp.sum(bank * bank, axis=0, keepdims=True))
    bn = bank / jnp.maximum(bn_n, 1e-8)
    sims = jax.lax.dot_general(bn, fn, (((0,), (0,)), ((), ())))
    smax = jnp.max(sims, axis=1, keepdims=True)
    iop = jax.lax.broadcasted_iota(jnp.int32, (K, P), 1)
    pp = jnp.min(jnp.where(sims == smax, iop, P), axis=1, keepdims=True)
    pp_ref[0] = pp


def _assemble_kernel(labels_ref, route_ref, vm_ref,
                     feat_ref, bank_ref, ac_ref,
                     out_ref,
                     alig_ref):
    s = pl.program_id(0)
    t = jnp.maximum(s - C, 0)
    b = t // 2
    ph = t % 2
    is_copy = s < C

    @pl.when(is_copy)
    def _copy():
        out_ref[...] = bank_ref[...]

    @pl.when(jnp.logical_not(is_copy) & (ph == 0))
    def _compute():
        fv = feat_ref[0]
        ac = ac_ref[0]
        iokp = jax.lax.broadcasted_iota(jnp.int32, (K, P), 1)
        oh_a = (iokp == ac).astype(jnp.float32)
        gathered = jax.lax.dot_general(
            fv, oh_a, (((1,), (1,)), ((), ())))
        alig = gathered * vm_ref[b].astype(jnp.float32)
        alig_ref[...] = alig

        upd = route_ref[b] < C

        @pl.when(upd)
        def _vals():
            rs = jnp.sum(jnp.maximum(fv, 0.0), axis=0, keepdims=True)
            iow = jax.lax.broadcasted_iota(jnp.int32, (P, MS), 0)
            iww = jax.lax.broadcasted_iota(jnp.int32, (P, MS), 1)
            colsel = ((iow % MS) == iww).astype(jnp.float32)
            nrm2 = jax.lax.dot_general(
                rs * rs, colsel, (((1,), (0,)), ((), ())))
            nrm_full = jax.lax.dot_general(
                nrm2, colsel, (((1,), (1,)), ((), ())))
            hm = rs / jnp.maximum(jnp.sqrt(nrm_full), 1e-12)

            io1 = jax.lax.broadcasted_iota(jnp.int32, (1, P), 1)
            iork = jax.lax.broadcasted_iota(jnp.int32, (K, P), 0)

            def body(k, carry):
                hm_cur, oh = carry
                mxv = jnp.max(hm_cur)
                pos = jnp.min(jnp.where(hm_cur == mxv, io1, P))
                hit = (iork == k) & (iokp == pos)
                oh = jnp.where(hit, 1.0, oh)
                hm_cur = jnp.where(io1 == pos, -jnp.inf, hm_cur)
                return hm_cur, oh

            _, oh_pick = jax.lax.fori_loop(
                0, K, body, (hm, jnp.zeros((K, P), jnp.float32)))
            vals = jax.lax.dot_general(
                fv, oh_pick, (((1,), (1,)), ((), ())))
            out_ref[0] = vals

        @pl.when(jnp.logical_not(upd))
        def _no_vals():
            out_ref[0] = alig

    @pl.when(jnp.logical_not(is_copy) & (ph == 1))
    def _aligned_out():
        out_ref[0] = alig_ref[...]


def _procrustes(src, dst):
    mu_s = src.mean(axis=0)
    mu_d = dst.mean(axis=0)
    sc = src - mu_s
    dc = dst - mu_d
    M = sc.T @ dc
    U, S, Vt = jnp.linalg.svd(M, full_matrices=False)
    R = U @ Vt
    scale = S.sum() / ((sc * sc).sum() + 1e-8)
    return scale * (sc @ R) + mu_d


@jax.jit
def kernel(scores, feat, feat_bank, bank_confidence, labels, bank_position):
    labels = labels.astype(jnp.int32)
    feat3 = feat.reshape(B, DIM, P)
    conf2 = bank_confidence.reshape(1, C)
    lab2 = labels.reshape(B, 1)

    route, fm = pl.pallas_call(
        _masks_kernel,
        out_shape=(
            jax.ShapeDtypeStruct((B, 1), jnp.int32),
            jax.ShapeDtypeStruct((B, 1), jnp.int32),
        ),
    )(scores, conf2, lab2)
    route1 = route.reshape(B)
    fm1 = fm.reshape(B)

    def feat_map_b(b, lab_r):
        return (b, 0, 0)

    def bank_map_b(b, lab_r):
        return (lab_r[b], 0, 0)

    pp = pl.pallas_call(
        _match_kernel,
        grid_spec=pltpu.PrefetchScalarGridSpec(
            num_scalar_prefetch=1,
            grid=(B,),
            in_specs=[
                pl.BlockSpec((1, DIM, P), feat_map_b),
                pl.BlockSpec((1, DIM, K), bank_map_b),
            ],
            out_specs=pl.BlockSpec((1, K, 1), lambda b, lab_r: (b, 0, 0)),
        ),
        out_shape=jax.ShapeDtypeStruct((B, K, 1), jnp.int32),
        compiler_params=pltpu.CompilerParams(
            dimension_semantics=("arbitrary",),
        ),
    )(labels, feat3, feat_bank)

    def tail(pp_b, label_b):
        pred_coord = jnp.stack([pp_b // MS, pp_b % MS], axis=1).astype(jnp.float32)
        sp = bank_position[label_b]
        st_coord = jnp.stack([sp // MS, sp % MS], axis=1).astype(jnp.float32)
        pred_coord = jax.lax.stop_gradient(pred_coord)
        st_coord = jax.lax.stop_gradient(st_coord)
        remap = _procrustes(pred_coord, st_coord)
        aligned = jnp.round(remap[:, 0]) * MS + jnp.round(remap[:, 1])
        valid = (pred_coord.sum() > 0) & (st_coord.sum() > 0)
        valid = valid & jnp.all(aligned >= 0) & jnp.all(aligned <= P - 1)
        valid = valid & (~jnp.any(jnp.isnan(aligned)))
        ac = jnp.clip(aligned, 0, P - 1).astype(jnp.int32)
        return ac, valid

    ac, valid = jax.vmap(tail)(pp[:, :, 0], labels)
    ac3 = ac.reshape(B, K, 1)
    vmask = fm1 * valid.astype(jnp.int32)

    def feat_map(s, lab_r, rt_r, vm_r):
        return (jnp.maximum(s - C, 0) // 2, 0, 0)

    def bank_map(s, lab_r, rt_r, vm_r):
        return (jnp.minimum(s, C - 1), 0, 0)

    def ac_map(s, lab_r, rt_r, vm_r):
        return (jnp.maximum(s - C, 0) // 2, 0, 0)

    def out_map(s, lab_r, rt_r, vm_r):
        t = jnp.maximum(s - C, 0)
        b = t // 2
        ph = t % 2
        samp_row = jnp.where(ph == 0, rt_r[b], C + b)
        return (jnp.where(s < C, s, samp_row), 0, 0)

    out = pl.pallas_call(
        _assemble_kernel,
        grid_spec=pltpu.PrefetchScalarGridSpec(
            num_scalar_prefetch=3,
            grid=(C + 2 * B,),
            in_specs=[
                pl.BlockSpec((1, DIM, P), feat_map),
                pl.BlockSpec((1, DIM, K), bank_map),
                pl.BlockSpec((1, K, 1), ac_map),
            ],
            out_specs=pl.BlockSpec((1, DIM, K), out_map),
            scratch_shapes=[pltpu.VMEM((DIM, K), jnp.float32)],
        ),
        out_shape=jax.ShapeDtypeStruct((C + B, DIM, K), jnp.float32),
        compiler_params=pltpu.CompilerParams(
            dimension_semantics=("arbitrary",),
        ),
    )(labels, route1, vmask, feat3, feat_bank, ac3)
    return out
